# baseline (speedup 1.0000x reference)
import jax
import jax.numpy as jnp
from jax import lax
from jax.experimental import pallas as pl

N = 10000
D = 128
NLAYERS = 64
BM = 2000


def _mm_body(x_ref, w_ref, o_ref):
    o_ref[...] = lax.dot_general(x_ref[...], w_ref[...], (((1,), (0,)), ((), ())),
                                 preferred_element_type=jnp.float32)


def _msg_matmul(x, w):
    return pl.pallas_call(
        _mm_body,
        grid=(N // BM,),
        in_specs=[pl.BlockSpec((BM, D), lambda i: (i, 0)),
                  pl.BlockSpec((D, D), lambda i: (0, 0))],
        out_specs=pl.BlockSpec((BM, D), lambda i: (i, 0)),
        out_shape=jax.ShapeDtypeStruct((N, D), jnp.float32),
    )(x, w)


def _gh_body(x_ref, w_ref, b_ref, o_ref):
    o_ref[...] = lax.dot_general(x_ref[...], w_ref[...], (((1,), (1,)), ((), ())),
                                 preferred_element_type=jnp.float32) + b_ref[...]


def _gh_proj(x, w_hh, b_hh):
    return pl.pallas_call(
        _gh_body,
        grid=(N // BM,),
        in_specs=[pl.BlockSpec((BM, D), lambda i: (i, 0)),
                  pl.BlockSpec((3 * D, D), lambda i: (0, 0)),
                  pl.BlockSpec((1, 3 * D), lambda i: (0, 0))],
        out_specs=pl.BlockSpec((BM, 3 * D), lambda i: (i, 0)),
        out_shape=jax.ShapeDtypeStruct((N, 3 * D), jnp.float32),
    )(x, w_hh, b_hh)


def _gates_body(agg_ref, gh_ref, x_ref, w_ref, b_ref, o_ref):
    gi = lax.dot_general(agg_ref[...], w_ref[...], (((1,), (1,)), ((), ())),
                         preferred_element_type=jnp.float32) + b_ref[...]
    gh = gh_ref[...]
    x = x_ref[...]
    r = jax.nn.sigmoid(gi[:, :D] + gh[:, :D])
    z = jax.nn.sigmoid(gi[:, D:2 * D] + gh[:, D:2 * D])
    n_ = jnp.tanh(gi[:, 2 * D:] + r * gh[:, 2 * D:])
    o_ref[...] = (1.0 - z) * n_ + z * x


def _gru_gates(agg, gh, x, w_ih, b_ih):
    return pl.pallas_call(
        _gates_body,
        grid=(N // BM,),
        in_specs=[pl.BlockSpec((BM, D), lambda i: (i, 0)),
                  pl.BlockSpec((BM, 3 * D), lambda i: (i, 0)),
                  pl.BlockSpec((BM, D), lambda i: (i, 0)),
                  pl.BlockSpec((3 * D, D), lambda i: (0, 0)),
                  pl.BlockSpec((1, 3 * D), lambda i: (0, 0))],
        out_specs=pl.BlockSpec((BM, D), lambda i: (i, 0)),
        out_shape=jax.ShapeDtypeStruct((N, D), jnp.float32),
    )(agg, gh, x, w_ih, b_ih)


def _final_body(x_ref, w_ref, b_ref, o_ref):
    h = jnp.maximum(x_ref[...], 0.0)
    o_ref[...] = lax.dot_general(h, w_ref[...], (((1,), (1,)), ((), ())),
                                 preferred_element_type=jnp.float32) + b_ref[...]


def kernel(obs, edge_index, ggc_weight, gru_w_ih, gru_w_hh, gru_b_ih, gru_b_hh,
           lin_w, lin_b):
    src = edge_index[0]
    dst = edge_index[1]
    bi = gru_b_ih.reshape(1, 3 * D)
    bh = gru_b_hh.reshape(1, 3 * D)

    def step(x, w):
        m = _msg_matmul(x, w)
        agg = jax.ops.segment_sum(m[src], dst, num_segments=N)
        gh = _gh_proj(x, gru_w_hh, bh)
        x2 = _gru_gates(agg, gh, x, gru_w_ih, bi)
        return x2, None

    x, _ = lax.scan(step, obs, ggc_weight)

    out = pl.pallas_call(
        _final_body,
        out_shape=jax.ShapeDtypeStruct((1, D), jnp.float32),
    )(x[N - 1:N], lin_w, lin_b.reshape(1, D))
    return out.reshape(D)


# speedup vs baseline: 4.0995x; 4.0995x over previous
import dataclasses
import functools

import numpy as np

import jax
import jax.numpy as jnp
from jax import lax
from jax.experimental import pallas as pl
from jax.experimental.pallas import tpu as pltpu
from jax.experimental.pallas import tpu_sc as plsc

N = 10000
D = 128
NLAYERS = 64
E = 160000
BM = 2000

NW = 32
EPT = 5120
NCK = EPT // 128
DUMP0 = 10000
PF0 = 10016
PL0 = 10048
NOUT = 10080

_RANGES = []
for _c in (0, 1):
    _pos = 0
    for _t in range(16):
        _w = 21 if _t < 14 else 20
        _pos2 = min(_pos + _w * 240, 80000)
        _RANGES.append((_c * 80000 + _pos, _c * 80000 + _pos2))
        _pos = _pos2
_STARTS = np.array([r[0] for r in _RANGES])
_ENDS = np.array([r[1] for r in _RANGES])
_LENS = _ENDS - _STARTS
_TILE_OF_POS = np.searchsorted(_ENDS, np.arange(E), side="right")
_IS_START = np.zeros(E, bool)
_IS_START[_STARTS] = True
_FLATIDX = _TILE_OF_POS * EPT + (np.arange(E) - _STARTS[_TILE_OF_POS])
_SRC_BASE = ((np.arange(NW * EPT) * 7) % N).astype(np.int32)
_ROW_BASE = np.broadcast_to(
    (DUMP0 + (np.arange(EPT) % 16)).astype(np.int32), (NW, EPT)).copy()
_PADLEN = (EPT - _LENS).astype(np.int32)


def _build_meta(src, dst):
    order = jnp.argsort(dst, stable=True)
    dst_s = dst[order].astype(jnp.int32)
    src_s = src[order].astype(jnp.int32)

    chg = jnp.concatenate([jnp.ones((1,), bool), dst_s[1:] != dst_s[:-1]])
    cut = chg | jnp.asarray(_IS_START)
    seg_id = jnp.cumsum(cut.astype(jnp.int32)) - 1
    nseg_tot = seg_id[-1] + 1
    seg_len = jax.ops.segment_sum(jnp.ones((E,), jnp.int32), seg_id, num_segments=E)
    seg_row = jnp.zeros((E,), jnp.int32).at[seg_id].set(dst_s)
    seg_tile = jnp.zeros((E,), jnp.int32).at[seg_id].set(jnp.asarray(_TILE_OF_POS, jnp.int32))
    nseg_per_tile = jax.ops.segment_sum(cut.astype(jnp.int32),
                                        jnp.asarray(_TILE_OF_POS, jnp.int32),
                                        num_segments=NW)
    first_seg = jnp.concatenate([jnp.zeros((1,), jnp.int32),
                                 jnp.cumsum(nseg_per_tile)[:-1].astype(jnp.int32)])
    k = jnp.arange(E)
    seg_local = k.astype(jnp.int32) - first_seg[seg_tile]
    valid = k < nseg_tot
    tile_idx = jnp.where(valid, seg_tile, NW)

    rowvals = jnp.where(
        seg_local == 0, PF0 + seg_tile,
        jnp.where(seg_local == nseg_per_tile[seg_tile] - 1, PL0 + seg_tile, seg_row))

    seglen = jnp.zeros((NW + 1, EPT), jnp.int32).at[tile_idx, seg_local].set(seg_len)
    segrow = jnp.asarray(np.concatenate([_ROW_BASE, np.zeros((1, EPT), np.int32)]))
    segrow = segrow.at[tile_idx, seg_local].set(rowvals)
    seglen = seglen.at[jnp.arange(NW), nseg_per_tile].set(jnp.asarray(_PADLEN))
    nsegp = ((nseg_per_tile + 1 + 127) // 128) * 128
    scal = jnp.broadcast_to(nsegp[:, None], (NW, 16))

    srcp = jnp.asarray(_SRC_BASE).at[jnp.asarray(_FLATIDX)].set(src_s)

    f_idx = dst_s[jnp.asarray(_STARTS)]
    l_idx = dst_s[jnp.asarray(_ENDS - 1)]
    has_edge = (jax.ops.segment_sum(jnp.ones((E,), jnp.int32), dst, num_segments=N)
                > 0)[:, None]
    return dict(
        srcp=srcp.reshape(NW, NCK, 128),
        seglen=seglen[:NW].reshape(NW, NCK, 8, 16),
        segrow=segrow[:NW].reshape(NW, NCK, 128),
        scal=scal, f_idx=f_idx, l_idx=l_idx, has_edge=has_edge,
        cont=(f_idx[1:] == l_idx[:-1])[:, None])


def _sc_segsum(m, meta):
    mesh = plsc.VectorSubcoreMesh(core_axis_name="c", subcore_axis_name="s")
    cp = pltpu.CompilerParams()
    if "needs_layout_passes" in pltpu.CompilerParams.__dataclass_fields__:
        cp = dataclasses.replace(cp, needs_layout_passes=False)

    @functools.partial(
        pl.kernel,
        out_type=jax.ShapeDtypeStruct((NOUT, D), jnp.float32),
        mesh=mesh,
        compiler_params=cp,
        scratch_types=[
            pltpu.VMEM((NCK, 128), jnp.int32),
            pltpu.VMEM((NCK, 128), jnp.int32),
            pltpu.VMEM((128, D), jnp.float32),
            pltpu.VMEM((128, D), jnp.float32),
            pltpu.VMEM((8, 16), jnp.int32),
            pltpu.VMEM((16,), jnp.int32),
        ],
    )
    def k(m_hbm, srcp_hbm, seglen_hbm, segrow_hbm, scal_hbm, out_hbm,
          srcv, segrowv, rows, stage, slen_v, scal_v):
        c = lax.axis_index("c")
        s = lax.axis_index("s")
        wid = s * 2 + c

        pltpu.sync_copy(srcp_hbm.at[wid], srcv)
        pltpu.sync_copy(segrow_hbm.at[wid], segrowv)
        pltpu.sync_copy(scal_hbm.at[wid], scal_v)
        nsegp = jnp.max(scal_v[...])
        lanes = lax.iota(jnp.int32, 16)

        def seg_body(i, ep):
            @pl.when((i & 127) == 0)
            def _():
                pltpu.sync_copy(seglen_hbm.at[wid, i >> 7], slen_v)

            grp = slen_v[(i & 127) >> 4, :]
            n = jnp.max(jnp.where(lanes == (i & 15), grp, 0))

            def edge_body(j, car):
                ep_ = car[0]

                @pl.when((ep_ & 127) == 0)
                def _():
                    pltpu.sync_copy(m_hbm.at[srcv.at[ep_ >> 7]], rows)

                off = ep_ & 127
                accs = tuple(car[1 + t] + rows[off, pl.ds(16 * t, 16)]
                             for t in range(8))
                return (ep_ + 1,) + accs

            zero = jnp.zeros((16,), jnp.float32)
            res = lax.fori_loop(0, n, edge_body, (ep,) + (zero,) * 8)
            si = i & 127
            for t in range(8):
                stage[si, pl.ds(16 * t, 16)] = res[1 + t]

            @pl.when(si == 127)
            def _():
                pltpu.sync_copy(stage, out_hbm.at[segrowv.at[i >> 7]])

            return res[0]

        lax.fori_loop(0, nsegp, seg_body, 0)

    return k(m, meta["srcp"], meta["seglen"], meta["segrow"], meta["scal"])


def _patch(out_ext, meta):
    base = jnp.where(meta["has_edge"], out_ext[:N], 0.0)
    pf = out_ext[PF0:PF0 + NW]
    pl_ = out_ext[PL0:PL0 + NW]
    lastv = jnp.concatenate(
        [jnp.where(meta["cont"], pl_[:-1] + pf[1:], pl_[:-1]), pl_[NW - 1:]], axis=0)
    base = base.at[meta["f_idx"]].set(pf)
    base = base.at[meta["l_idx"]].set(lastv)
    return base


def _mm_body(x_ref, w_ref, o_ref):
    o_ref[...] = lax.dot_general(x_ref[...], w_ref[...], (((1,), (0,)), ((), ())),
                                 preferred_element_type=jnp.float32)


def _msg_matmul(x, w):
    return pl.pallas_call(
        _mm_body,
        grid=(N // BM,),
        in_specs=[pl.BlockSpec((BM, D), lambda i: (i, 0)),
                  pl.BlockSpec((D, D), lambda i: (0, 0))],
        out_specs=pl.BlockSpec((BM, D), lambda i: (i, 0)),
        out_shape=jax.ShapeDtypeStruct((N, D), jnp.float32),
    )(x, w)


def _gh_body(x_ref, w_ref, b_ref, o_ref):
    o_ref[...] = lax.dot_general(x_ref[...], w_ref[...], (((1,), (1,)), ((), ())),
                                 preferred_element_type=jnp.float32) + b_ref[...]


def _gh_proj(x, w_hh, b_hh):
    return pl.pallas_call(
        _gh_body,
        grid=(N // BM,),
        in_specs=[pl.BlockSpec((BM, D), lambda i: (i, 0)),
                  pl.BlockSpec((3 * D, D), lambda i: (0, 0)),
                  pl.BlockSpec((1, 3 * D), lambda i: (0, 0))],
        out_specs=pl.BlockSpec((BM, 3 * D), lambda i: (i, 0)),
        out_shape=jax.ShapeDtypeStruct((N, 3 * D), jnp.float32),
    )(x, w_hh, b_hh)


def _gates_body(agg_ref, gh_ref, x_ref, w_ref, b_ref, o_ref):
    gi = lax.dot_general(agg_ref[...], w_ref[...], (((1,), (1,)), ((), ())),
                         preferred_element_type=jnp.float32) + b_ref[...]
    gh = gh_ref[...]
    x = x_ref[...]
    r = jax.nn.sigmoid(gi[:, :D] + gh[:, :D])
    z = jax.nn.sigmoid(gi[:, D:2 * D] + gh[:, D:2 * D])
    n_ = jnp.tanh(gi[:, 2 * D:] + r * gh[:, 2 * D:])
    o_ref[...] = (1.0 - z) * n_ + z * x


def _gru_gates(agg, gh, x, w_ih, b_ih):
    return pl.pallas_call(
        _gates_body,
        grid=(N // BM,),
        in_specs=[pl.BlockSpec((BM, D), lambda i: (i, 0)),
                  pl.BlockSpec((BM, 3 * D), lambda i: (i, 0)),
                  pl.BlockSpec((BM, D), lambda i: (i, 0)),
                  pl.BlockSpec((3 * D, D), lambda i: (0, 0)),
                  pl.BlockSpec((1, 3 * D), lambda i: (0, 0))],
        out_specs=pl.BlockSpec((BM, D), lambda i: (i, 0)),
        out_shape=jax.ShapeDtypeStruct((N, D), jnp.float32),
    )(agg, gh, x, w_ih, b_ih)


def _final_body(x_ref, w_ref, b_ref, o_ref):
    h = jnp.maximum(x_ref[...], 0.0)
    o_ref[...] = lax.dot_general(h, w_ref[...], (((1,), (1,)), ((), ())),
                                 preferred_element_type=jnp.float32) + b_ref[...]


def kernel(obs, edge_index, ggc_weight, gru_w_ih, gru_w_hh, gru_b_ih, gru_b_hh,
           lin_w, lin_b):
    src = edge_index[0]
    dst = edge_index[1]
    bi = gru_b_ih.reshape(1, 3 * D)
    bh = gru_b_hh.reshape(1, 3 * D)
    meta = _build_meta(src, dst)

    def step(x, w):
        m = _msg_matmul(x, w)
        out_ext = _sc_segsum(m, meta)
        gh = _gh_proj(x, gru_w_hh, bh)
        agg = _patch(out_ext, meta)
        x2 = _gru_gates(agg, gh, x, gru_w_ih, bi)
        return x2, None

    x, _ = lax.scan(step, obs, ggc_weight)

    out = pl.pallas_call(
        _final_body,
        out_shape=jax.ShapeDtypeStruct((1, D), jnp.float32),
    )(x[N - 1:N], lin_w, lin_b.reshape(1, D))
    return out.reshape(D)


# speedup vs baseline: 5.2523x; 1.2812x over previous
import dataclasses
import functools

import numpy as np

import jax
import jax.numpy as jnp
from jax import lax
from jax.experimental import pallas as pl
from jax.experimental.pallas import tpu as pltpu
from jax.experimental.pallas import tpu_sc as plsc

N = 10000
D = 128
NLAYERS = 64
E = 160000
BM = 2000

NW = 32
EPT = 5120
NCK = EPT // 128
DUMP0 = 10000
PF0 = 10016
PL0 = 10048
NOUT = 10080

_RANGES = []
for _c in (0, 1):
    _pos = 0
    for _t in range(16):
        _w = 21 if _t < 14 else 20
        _pos2 = min(_pos + _w * 240, 80000)
        _RANGES.append((_c * 80000 + _pos, _c * 80000 + _pos2))
        _pos = _pos2
_STARTS = np.array([r[0] for r in _RANGES])
_ENDS = np.array([r[1] for r in _RANGES])
_LENS = _ENDS - _STARTS
_TILE_OF_POS = np.searchsorted(_ENDS, np.arange(E), side="right")
_IS_START = np.zeros(E, bool)
_IS_START[_STARTS] = True
_FLATIDX = _TILE_OF_POS * EPT + (np.arange(E) - _STARTS[_TILE_OF_POS])
_SRC_BASE = ((np.arange(NW * EPT) * 7) % N).astype(np.int32)
_ROW_BASE = np.broadcast_to(
    (DUMP0 + (np.arange(EPT) % 16)).astype(np.int32), (NW, EPT)).copy()
_PADLEN = (EPT - _LENS).astype(np.int32)


def _build_meta(src, dst):
    order = jnp.argsort(dst, stable=True)
    dst_s = dst[order].astype(jnp.int32)
    src_s = src[order].astype(jnp.int32)

    chg = jnp.concatenate([jnp.ones((1,), bool), dst_s[1:] != dst_s[:-1]])
    cut = chg | jnp.asarray(_IS_START)
    seg_id = jnp.cumsum(cut.astype(jnp.int32)) - 1
    nseg_tot = seg_id[-1] + 1
    seg_len = jax.ops.segment_sum(jnp.ones((E,), jnp.int32), seg_id, num_segments=E)
    seg_row = jnp.zeros((E,), jnp.int32).at[seg_id].set(dst_s)
    seg_tile = jnp.zeros((E,), jnp.int32).at[seg_id].set(jnp.asarray(_TILE_OF_POS, jnp.int32))
    nseg_per_tile = jax.ops.segment_sum(cut.astype(jnp.int32),
                                        jnp.asarray(_TILE_OF_POS, jnp.int32),
                                        num_segments=NW)
    first_seg = jnp.concatenate([jnp.zeros((1,), jnp.int32),
                                 jnp.cumsum(nseg_per_tile)[:-1].astype(jnp.int32)])
    k = jnp.arange(E)
    seg_local = k.astype(jnp.int32) - first_seg[seg_tile]
    valid = k < nseg_tot
    tile_idx = jnp.where(valid, seg_tile, NW)

    rowvals = jnp.where(
        seg_local == 0, PF0 + seg_tile,
        jnp.where(seg_local == nseg_per_tile[seg_tile] - 1, PL0 + seg_tile, seg_row))

    seglen = jnp.zeros((NW + 1, EPT), jnp.int32).at[tile_idx, seg_local].set(seg_len)
    segrow = jnp.asarray(np.concatenate([_ROW_BASE, np.zeros((1, EPT), np.int32)]))
    segrow = segrow.at[tile_idx, seg_local].set(rowvals)
    seglen = seglen.at[jnp.arange(NW), nseg_per_tile].set(jnp.asarray(_PADLEN))
    nsegp = ((nseg_per_tile + 1 + 127) // 128) * 128
    scal = jnp.broadcast_to(nsegp[:, None], (NW, 16))

    srcp = jnp.asarray(_SRC_BASE).at[jnp.asarray(_FLATIDX)].set(src_s)

    f_idx = dst_s[jnp.asarray(_STARTS)]
    l_idx = dst_s[jnp.asarray(_ENDS - 1)]
    has_edge = (jax.ops.segment_sum(jnp.ones((E,), jnp.int32), dst, num_segments=N)
                > 0)[:, None]
    return dict(
        srcp=srcp.reshape(NW, NCK, 128),
        seglen=seglen[:NW].reshape(NW, NCK, 8, 16),
        segrow=segrow[:NW].reshape(NW, NCK, 128),
        scal=scal, f_idx=f_idx, l_idx=l_idx, has_edge=has_edge,
        cont=(f_idx[1:] == l_idx[:-1])[:, None])


def _sc_segsum(m, meta):
    mesh = plsc.VectorSubcoreMesh(core_axis_name="c", subcore_axis_name="s")
    cp = pltpu.CompilerParams()
    if "needs_layout_passes" in pltpu.CompilerParams.__dataclass_fields__:
        cp = dataclasses.replace(cp, needs_layout_passes=False)

    @functools.partial(
        pl.kernel,
        out_type=jax.ShapeDtypeStruct((NOUT, D), jnp.float32),
        mesh=mesh,
        compiler_params=cp,
        scratch_types=[
            pltpu.VMEM((NCK, 128), jnp.int32),
            pltpu.VMEM((NCK, 128), jnp.int32),
            pltpu.VMEM((4, 128, D), jnp.float32),
            pltpu.VMEM((2, 128, D), jnp.float32),
            pltpu.VMEM((8, 16), jnp.int32),
            pltpu.VMEM((16,), jnp.int32),
            pltpu.SemaphoreType.DMA((4,)),
            pltpu.SemaphoreType.DMA((2,)),
        ],
    )
    def k(m_hbm, srcp_hbm, seglen_hbm, segrow_hbm, scal_hbm, out_hbm,
          srcv, segrowv, rows, stage, slen_v, scal_v, gsem, ssem):
        c = lax.axis_index("c")
        s = lax.axis_index("s")
        wid = s * 2 + c

        pltpu.sync_copy(srcp_hbm.at[wid], srcv)
        pltpu.sync_copy(segrow_hbm.at[wid], segrowv)
        pltpu.sync_copy(scal_hbm.at[wid], scal_v)
        nsegp = jnp.max(scal_v[...])
        lanes = lax.iota(jnp.int32, 16)

        for k0 in range(3):
            pltpu.async_copy(m_hbm.at[srcv.at[k0]], rows.at[k0], gsem.at[k0])

        def seg_body(i, ep):
            kc = i >> 7

            @pl.when((i & 127) == 0)
            def _():
                pltpu.sync_copy(seglen_hbm.at[wid, kc], slen_v)
                @pl.when(kc >= 2)
                def _():
                    pltpu.make_async_copy(stage.at[kc & 1],
                                          out_hbm.at[segrowv.at[kc]],
                                          ssem.at[kc & 1]).wait()

            grp = slen_v[(i & 127) >> 4, :]
            n = jnp.max(jnp.where(lanes == (i & 15), grp, 0))

            def edge_body(j, car):
                ep_ = car[0]
                ck = ep_ >> 7
                slot = ck & 3

                @pl.when((ep_ & 127) == 0)
                def _():
                    pltpu.make_async_copy(m_hbm.at[srcv.at[ck]],
                                          rows.at[slot], gsem.at[slot]).wait()
                    @pl.when(ck + 3 <= NCK - 1)
                    def _():
                        pltpu.async_copy(m_hbm.at[srcv.at[ck + 3]],
                                         rows.at[(ck + 3) & 3],
                                         gsem.at[(ck + 3) & 3])

                off = ep_ & 127
                accs = tuple(car[1 + t] + rows[slot, off, pl.ds(16 * t, 16)]
                             for t in range(8))
                return (ep_ + 1,) + accs

            zero = jnp.zeros((16,), jnp.float32)
            res = lax.fori_loop(0, n, edge_body, (ep,) + (zero,) * 8)
            si = i & 127
            for t in range(8):
                stage[kc & 1, si, pl.ds(16 * t, 16)] = res[1 + t]

            @pl.when(si == 127)
            def _():
                pltpu.async_copy(stage.at[kc & 1], out_hbm.at[segrowv.at[kc]],
                                 ssem.at[kc & 1])

            return res[0]

        lax.fori_loop(0, nsegp, seg_body, 0)

        nbuf = nsegp >> 7

        @pl.when(nbuf >= 2)
        def _():
            pltpu.make_async_copy(stage.at[nbuf & 1], out_hbm.at[segrowv.at[0]],
                                  ssem.at[nbuf & 1]).wait()

        pltpu.make_async_copy(stage.at[(nbuf - 1) & 1], out_hbm.at[segrowv.at[0]],
                              ssem.at[(nbuf - 1) & 1]).wait()

    return k(m, meta["srcp"], meta["seglen"], meta["segrow"], meta["scal"])


def _patch(out_ext, meta):
    base = jnp.where(meta["has_edge"], out_ext[:N], 0.0)
    pf = out_ext[PF0:PF0 + NW]
    pl_ = out_ext[PL0:PL0 + NW]
    lastv = jnp.concatenate(
        [jnp.where(meta["cont"], pl_[:-1] + pf[1:], pl_[:-1]), pl_[NW - 1:]], axis=0)
    base = base.at[meta["f_idx"]].set(pf)
    base = base.at[meta["l_idx"]].set(lastv)
    return base


def _mm_body(x_ref, w_ref, o_ref):
    o_ref[...] = lax.dot_general(x_ref[...], w_ref[...], (((1,), (0,)), ((), ())),
                                 preferred_element_type=jnp.float32)


def _msg_matmul(x, w):
    return pl.pallas_call(
        _mm_body,
        grid=(N // BM,),
        in_specs=[pl.BlockSpec((BM, D), lambda i: (i, 0)),
                  pl.BlockSpec((D, D), lambda i: (0, 0))],
        out_specs=pl.BlockSpec((BM, D), lambda i: (i, 0)),
        out_shape=jax.ShapeDtypeStruct((N, D), jnp.float32),
    )(x, w)


def _gh_body(x_ref, w_ref, b_ref, o_ref):
    o_ref[...] = lax.dot_general(x_ref[...], w_ref[...], (((1,), (1,)), ((), ())),
                                 preferred_element_type=jnp.float32) + b_ref[...]


def _gh_proj(x, w_hh, b_hh):
    return pl.pallas_call(
        _gh_body,
        grid=(N // BM,),
        in_specs=[pl.BlockSpec((BM, D), lambda i: (i, 0)),
                  pl.BlockSpec((3 * D, D), lambda i: (0, 0)),
                  pl.BlockSpec((1, 3 * D), lambda i: (0, 0))],
        out_specs=pl.BlockSpec((BM, 3 * D), lambda i: (i, 0)),
        out_shape=jax.ShapeDtypeStruct((N, 3 * D), jnp.float32),
    )(x, w_hh, b_hh)


def _gates_body(agg_ref, gh_ref, x_ref, w_ref, b_ref, o_ref):
    gi = lax.dot_general(agg_ref[...], w_ref[...], (((1,), (1,)), ((), ())),
                         preferred_element_type=jnp.float32) + b_ref[...]
    gh = gh_ref[...]
    x = x_ref[...]
    r = jax.nn.sigmoid(gi[:, :D] + gh[:, :D])
    z = jax.nn.sigmoid(gi[:, D:2 * D] + gh[:, D:2 * D])
    n_ = jnp.tanh(gi[:, 2 * D:] + r * gh[:, 2 * D:])
    o_ref[...] = (1.0 - z) * n_ + z * x


def _gru_gates(agg, gh, x, w_ih, b_ih):
    return pl.pallas_call(
        _gates_body,
        grid=(N // BM,),
        in_specs=[pl.BlockSpec((BM, D), lambda i: (i, 0)),
                  pl.BlockSpec((BM, 3 * D), lambda i: (i, 0)),
                  pl.BlockSpec((BM, D), lambda i: (i, 0)),
                  pl.BlockSpec((3 * D, D), lambda i: (0, 0)),
                  pl.BlockSpec((1, 3 * D), lambda i: (0, 0))],
        out_specs=pl.BlockSpec((BM, D), lambda i: (i, 0)),
        out_shape=jax.ShapeDtypeStruct((N, D), jnp.float32),
    )(agg, gh, x, w_ih, b_ih)


def _final_body(x_ref, w_ref, b_ref, o_ref):
    h = jnp.maximum(x_ref[...], 0.0)
    o_ref[...] = lax.dot_general(h, w_ref[...], (((1,), (1,)), ((), ())),
                                 preferred_element_type=jnp.float32) + b_ref[...]


def kernel(obs, edge_index, ggc_weight, gru_w_ih, gru_w_hh, gru_b_ih, gru_b_hh,
           lin_w, lin_b):
    src = edge_index[0]
    dst = edge_index[1]
    bi = gru_b_ih.reshape(1, 3 * D)
    bh = gru_b_hh.reshape(1, 3 * D)
    meta = _build_meta(src, dst)

    def step(x, w):
        m = _msg_matmul(x, w)
        out_ext = _sc_segsum(m, meta)
        gh = _gh_proj(x, gru_w_hh, bh)
        agg = _patch(out_ext, meta)
        x2 = _gru_gates(agg, gh, x, gru_w_ih, bi)
        return x2, None

    x, _ = lax.scan(step, obs, ggc_weight)

    out = pl.pallas_call(
        _final_body,
        out_shape=jax.ShapeDtypeStruct((1, D), jnp.float32),
    )(x[N - 1:N], lin_w, lin_b.reshape(1, D))
    return out.reshape(D)


# speedup vs baseline: 5.6236x; 1.0707x over previous
import dataclasses
import functools

import numpy as np

import jax
import jax.numpy as jnp
from jax import lax
from jax.experimental import pallas as pl
from jax.experimental.pallas import tpu as pltpu
from jax.experimental.pallas import tpu_sc as plsc

N = 10000
D = 128
NLAYERS = 64
E = 160000
BM = 2000

NW = 32
EPT = 5120
NCK = EPT // 128
DUMP0 = 10000
PF0 = 10016
PL0 = 10048
NOUT = 10080

_RANGES = []
for _c in (0, 1):
    _pos = 0
    for _t in range(16):
        _w = 21 if _t < 14 else 20
        _pos2 = min(_pos + _w * 240, 80000)
        _RANGES.append((_c * 80000 + _pos, _c * 80000 + _pos2))
        _pos = _pos2
_STARTS = np.array([r[0] for r in _RANGES])
_ENDS = np.array([r[1] for r in _RANGES])
_LENS = _ENDS - _STARTS
_TILE_OF_POS = np.searchsorted(_ENDS, np.arange(E), side="right")
_IS_START = np.zeros(E, bool)
_IS_START[_STARTS] = True
_FLATIDX = _TILE_OF_POS * EPT + (np.arange(E) - _STARTS[_TILE_OF_POS])
_SRC_BASE = ((np.arange(NW * EPT) * 7) % N).astype(np.int32)
_ROW_BASE = np.broadcast_to(
    (DUMP0 + (np.arange(EPT) % 16)).astype(np.int32), (NW, EPT)).copy()
_PADLEN = (EPT - _LENS).astype(np.int32)


def _build_meta(src, dst):
    order = jnp.argsort(dst, stable=True)
    dst_s = dst[order].astype(jnp.int32)
    src_s = src[order].astype(jnp.int32)

    chg = jnp.concatenate([jnp.ones((1,), bool), dst_s[1:] != dst_s[:-1]])
    cut = chg | jnp.asarray(_IS_START)
    seg_id = jnp.cumsum(cut.astype(jnp.int32)) - 1
    nseg_tot = seg_id[-1] + 1
    seg_len = jax.ops.segment_sum(jnp.ones((E,), jnp.int32), seg_id, num_segments=E)
    seg_row = jnp.zeros((E,), jnp.int32).at[seg_id].set(dst_s)
    seg_tile = jnp.zeros((E,), jnp.int32).at[seg_id].set(jnp.asarray(_TILE_OF_POS, jnp.int32))
    nseg_per_tile = jax.ops.segment_sum(cut.astype(jnp.int32),
                                        jnp.asarray(_TILE_OF_POS, jnp.int32),
                                        num_segments=NW)
    first_seg = jnp.concatenate([jnp.zeros((1,), jnp.int32),
                                 jnp.cumsum(nseg_per_tile)[:-1].astype(jnp.int32)])
    k = jnp.arange(E)
    seg_local = k.astype(jnp.int32) - first_seg[seg_tile]
    valid = k < nseg_tot
    tile_idx = jnp.where(valid, seg_tile, NW)

    rowvals = jnp.where(
        seg_local == 0, PF0 + seg_tile,
        jnp.where(seg_local == nseg_per_tile[seg_tile] - 1, PL0 + seg_tile, seg_row))

    seglen = jnp.zeros((NW + 1, EPT), jnp.int32).at[tile_idx, seg_local].set(seg_len)
    segrow = jnp.asarray(np.concatenate([_ROW_BASE, np.zeros((1, EPT), np.int32)]))
    segrow = segrow.at[tile_idx, seg_local].set(rowvals)
    seglen = seglen.at[jnp.arange(NW), nseg_per_tile].set(jnp.asarray(_PADLEN))
    nsegp = ((nseg_per_tile + 1 + 127) // 128) * 128
    scal = jnp.broadcast_to(nsegp[:, None], (NW, 16))

    srcp = jnp.asarray(_SRC_BASE).at[jnp.asarray(_FLATIDX)].set(src_s)

    f_idx = dst_s[jnp.asarray(_STARTS)]
    l_idx = dst_s[jnp.asarray(_ENDS - 1)]
    has_edge = (jax.ops.segment_sum(jnp.ones((E,), jnp.int32), dst, num_segments=N)
                > 0)[:, None]
    return dict(
        srcp=srcp.reshape(NW, NCK, 128),
        seglen=seglen[:NW].reshape(NW, NCK, 8, 16),
        segrow=segrow[:NW].reshape(NW, NCK, 128),
        scal=scal, f_idx=f_idx, l_idx=l_idx, has_edge=has_edge,
        cont=(f_idx[1:] == l_idx[:-1])[:, None])


def _sc_segsum(m, meta):
    mesh = plsc.VectorSubcoreMesh(core_axis_name="c", subcore_axis_name="s")
    cp = pltpu.CompilerParams()
    if "needs_layout_passes" in pltpu.CompilerParams.__dataclass_fields__:
        cp = dataclasses.replace(cp, needs_layout_passes=False)

    @functools.partial(
        pl.kernel,
        out_type=jax.ShapeDtypeStruct((NOUT, D), jnp.float32),
        mesh=mesh,
        compiler_params=cp,
        scratch_types=[
            pltpu.VMEM((NCK, 128), jnp.int32),
            pltpu.VMEM((NCK, 128), jnp.int32),
            pltpu.VMEM((4, 128, D), jnp.float32),
            pltpu.VMEM((2, 128, D), jnp.float32),
            pltpu.VMEM((8, 16), jnp.int32),
            pltpu.VMEM((16,), jnp.int32),
            pltpu.SemaphoreType.DMA((4,)),
            pltpu.SemaphoreType.DMA((2,)),
        ],
    )
    def k(m_hbm, srcp_hbm, seglen_hbm, segrow_hbm, scal_hbm, out_hbm,
          srcv, segrowv, rows, stage, slen_v, scal_v, gsem, ssem):
        c = lax.axis_index("c")
        s = lax.axis_index("s")
        wid = s * 2 + c

        pltpu.sync_copy(srcp_hbm.at[wid], srcv)
        pltpu.sync_copy(segrow_hbm.at[wid], segrowv)
        pltpu.sync_copy(scal_hbm.at[wid], scal_v)
        nsegp = jnp.max(scal_v[...])
        lanes = lax.iota(jnp.int32, 16)

        for k0 in range(3):
            pltpu.async_copy(m_hbm.at[srcv.at[k0]], rows.at[k0], gsem.at[k0])

        def seg_body(i, ep):
            kc = i >> 7

            @pl.when((i & 127) == 0)
            def _():
                pltpu.sync_copy(seglen_hbm.at[wid, kc], slen_v)
                @pl.when(kc >= 2)
                def _():
                    pltpu.make_async_copy(stage.at[kc & 1],
                                          out_hbm.at[segrowv.at[kc]],
                                          ssem.at[kc & 1]).wait()

            grp = slen_v[(i & 127) >> 4, :]
            n = jnp.max(jnp.where(lanes == (i & 15), grp, 0))

            def edge_body(j, car):
                ep_ = car[0]
                ck = ep_ >> 7
                slot = ck & 3

                @pl.when((ep_ & 127) == 0)
                def _():
                    pltpu.make_async_copy(m_hbm.at[srcv.at[ck]],
                                          rows.at[slot], gsem.at[slot]).wait()
                    @pl.when(ck + 3 <= NCK - 1)
                    def _():
                        pltpu.async_copy(m_hbm.at[srcv.at[ck + 3]],
                                         rows.at[(ck + 3) & 3],
                                         gsem.at[(ck + 3) & 3])

                off = ep_ & 127
                accs = tuple(car[1 + t] + rows[slot, off, pl.ds(16 * t, 16)]
                             for t in range(8))
                return (ep_ + 1,) + accs

            zero = jnp.zeros((16,), jnp.float32)
            res = lax.fori_loop(0, n, edge_body, (ep,) + (zero,) * 8)
            si = i & 127
            for t in range(8):
                stage[kc & 1, si, pl.ds(16 * t, 16)] = res[1 + t]

            @pl.when(si == 127)
            def _():
                pltpu.async_copy(stage.at[kc & 1], out_hbm.at[segrowv.at[kc]],
                                 ssem.at[kc & 1])

            return res[0]

        lax.fori_loop(0, nsegp, seg_body, 0)

        nbuf = nsegp >> 7

        @pl.when(nbuf >= 2)
        def _():
            pltpu.make_async_copy(stage.at[nbuf & 1], out_hbm.at[segrowv.at[0]],
                                  ssem.at[nbuf & 1]).wait()

        pltpu.make_async_copy(stage.at[(nbuf - 1) & 1], out_hbm.at[segrowv.at[0]],
                              ssem.at[(nbuf - 1) & 1]).wait()

    return k(m, meta["srcp"], meta["seglen"], meta["segrow"], meta["scal"])


def _patch(out_ext, meta):
    base = jnp.where(meta["has_edge"], out_ext[:N], 0.0)
    pf = out_ext[PF0:PF0 + NW]
    pl_ = out_ext[PL0:PL0 + NW]
    lastv = jnp.concatenate(
        [jnp.where(meta["cont"], pl_[:-1] + pf[1:], pl_[:-1]), pl_[NW - 1:]], axis=0)
    base = base.at[meta["f_idx"]].set(pf)
    base = base.at[meta["l_idx"]].set(lastv)
    return base


def _mm_body(x_ref, w_ref, o_ref):
    o_ref[...] = lax.dot_general(x_ref[...], w_ref[...], (((1,), (0,)), ((), ())),
                                 preferred_element_type=jnp.float32)


def _msg_matmul(x, w):
    return pl.pallas_call(
        _mm_body,
        grid=(N // BM,),
        in_specs=[pl.BlockSpec((BM, D), lambda i: (i, 0)),
                  pl.BlockSpec((D, D), lambda i: (0, 0))],
        out_specs=pl.BlockSpec((BM, D), lambda i: (i, 0)),
        out_shape=jax.ShapeDtypeStruct((N, D), jnp.float32),
    )(x, w)


def _gh_body(x_ref, w_ref, b_ref, o_ref):
    o_ref[...] = lax.dot_general(x_ref[...], w_ref[...], (((1,), (1,)), ((), ())),
                                 preferred_element_type=jnp.float32) + b_ref[...]


def _gh_proj(x, w_hh, b_hh):
    return pl.pallas_call(
        _gh_body,
        grid=(N // BM,),
        in_specs=[pl.BlockSpec((BM, D), lambda i: (i, 0)),
                  pl.BlockSpec((3 * D, D), lambda i: (0, 0)),
                  pl.BlockSpec((1, 3 * D), lambda i: (0, 0))],
        out_specs=pl.BlockSpec((BM, 3 * D), lambda i: (i, 0)),
        out_shape=jax.ShapeDtypeStruct((N, 3 * D), jnp.float32),
    )(x, w_hh, b_hh)


def _gates_body(agg_ref, gh_ref, x_ref, w_ref, b_ref, wn_ref, o_ref, m_ref):
    gi = lax.dot_general(agg_ref[...], w_ref[...], (((1,), (1,)), ((), ())),
                         preferred_element_type=jnp.float32) + b_ref[...]
    gh = gh_ref[...]
    x = x_ref[...]
    r = jax.nn.sigmoid(gi[:, :D] + gh[:, :D])
    z = jax.nn.sigmoid(gi[:, D:2 * D] + gh[:, D:2 * D])
    n_ = jnp.tanh(gi[:, 2 * D:] + r * gh[:, 2 * D:])
    x2 = (1.0 - z) * n_ + z * x
    o_ref[...] = x2
    m_ref[...] = lax.dot_general(x2, wn_ref[...], (((1,), (0,)), ((), ())),
                                 preferred_element_type=jnp.float32)


def _gru_gates(agg, gh, x, w_ih, b_ih, w_next):
    return pl.pallas_call(
        _gates_body,
        grid=(N // BM,),
        in_specs=[pl.BlockSpec((BM, D), lambda i: (i, 0)),
                  pl.BlockSpec((BM, 3 * D), lambda i: (i, 0)),
                  pl.BlockSpec((BM, D), lambda i: (i, 0)),
                  pl.BlockSpec((3 * D, D), lambda i: (0, 0)),
                  pl.BlockSpec((1, 3 * D), lambda i: (0, 0)),
                  pl.BlockSpec((D, D), lambda i: (0, 0))],
        out_specs=[pl.BlockSpec((BM, D), lambda i: (i, 0)),
                   pl.BlockSpec((BM, D), lambda i: (i, 0))],
        out_shape=[jax.ShapeDtypeStruct((N, D), jnp.float32),
                   jax.ShapeDtypeStruct((N, D), jnp.float32)],
    )(agg, gh, x, w_ih, b_ih, w_next)


def _final_body(x_ref, w_ref, b_ref, o_ref):
    h = jnp.maximum(x_ref[...], 0.0)
    o_ref[...] = lax.dot_general(h, w_ref[...], (((1,), (1,)), ((), ())),
                                 preferred_element_type=jnp.float32) + b_ref[...]


def kernel(obs, edge_index, ggc_weight, gru_w_ih, gru_w_hh, gru_b_ih, gru_b_hh,
           lin_w, lin_b):
    src = edge_index[0]
    dst = edge_index[1]
    bi = gru_b_ih.reshape(1, 3 * D)
    bh = gru_b_hh.reshape(1, 3 * D)
    meta = _build_meta(src, dst)

    m0 = _msg_matmul(obs, ggc_weight[0])
    w_next = jnp.concatenate([ggc_weight[1:], ggc_weight[:1]], axis=0)

    def step(carry, wn):
        x, m = carry
        out_ext = _sc_segsum(m, meta)
        gh = _gh_proj(x, gru_w_hh, bh)
        agg = _patch(out_ext, meta)
        x2, m2 = _gru_gates(agg, gh, x, gru_w_ih, bi, wn)
        return (x2, m2), None

    (x, _), _ = lax.scan(step, (obs, m0), w_next)

    out = pl.pallas_call(
        _final_body,
        out_shape=jax.ShapeDtypeStruct((1, D), jnp.float32),
    )(x[N - 1:N], lin_w, lin_b.reshape(1, D))
    return out.reshape(D)
